# baseline (speedup 1.0000x reference)
import dataclasses
import functools

import numpy as np
import jax
import jax.numpy as jnp
from jax import lax
from jax.experimental import pallas as pl
from jax.experimental.pallas import tpu as pltpu
from jax.experimental.pallas import tpu_sc as plsc

_BUCKET_SIZE = 0.301
_N = 200
_NP = 208
_BB = 16
_BSC = 32
_SC_NC = 2
_SC_NS = 16
_SC_NW = _SC_NC * _SC_NS


def _build_bucket_tables():
    bs = np.float32(_BUCKET_SIZE)

    def bucket_f32(x32):
        return np.trunc(np.log(np.asarray(x32, np.float32)) / bs).astype(np.int32)

    base = np.zeros(32, np.int32)
    th = np.full((3, 32), 0x7FFFFFFF, np.int32)
    for e in range(31):
        base[e] = bucket_f32(np.float32(2.0 ** e))
        for k in range(1, 4):
            target = int(base[e]) + k
            m = np.exp(0.301 * target) / 2.0 ** e
            if m >= 2.0:
                continue
            mb0 = int(np.floor((m - 1.0) * 2 ** 23))
            lo, hi = max(mb0 - 64, 0), min(mb0 + 64, 2 ** 23 - 1)
            mbs = np.arange(lo, hi + 1, dtype=np.int64)
            vals = (((e + 127) << 23) | mbs).astype(np.int32).view(np.float32)
            idx = np.nonzero(bucket_f32(vals) >= target)[0]
            if len(idx):
                th[k - 1, e] = int(mbs[idx[0]])
    return base, th.reshape(-1)


_BASE_TBL, _TH_TBL = _build_bucket_tables()


def _pos_body(pos_ref, out_ref):
    n = _N
    i = jax.lax.broadcasted_iota(jnp.int32, (n, n), 0)
    j = jax.lax.broadcasted_iota(jnp.int32, (n, n), 1)
    p = (n - 1) + j - i
    hi = p >> 7
    lo = p & 127
    posv = jnp.zeros((n, n), jnp.float32)
    for c in range(4):
        chunk = jnp.broadcast_to(pos_ref[0:1, c * 128:(c + 1) * 128], (n, 128))
        g = jnp.take_along_axis(chunk, lo, axis=1, mode="promise_in_bounds")
        posv = jnp.where(hi == c, g, posv)
    out_ref[0] = posv


def _main_body(t_ref, s_ref, tsw_ref, pos_ref, sc_ref, out_ref):
    n = _N
    nsteps_tc = (1024 - _BSC) // _BB

    @pl.when(pl.program_id(0) < nsteps_tc)
    def _compute():
        table = jnp.broadcast_to(tsw_ref[0:1, :], (n, 128))
        pos = pos_ref[0]
        for k in range(_BB):
            t = t_ref[k]
            s = s_ref[k]
            diff = s - t
            x = jnp.maximum(jnp.abs(diff.astype(jnp.float32)), 1.0)
            b = (jnp.log(x) / _BUCKET_SIZE).astype(jnp.int32)
            tb = jnp.take_along_axis(table, b, axis=1, mode="promise_in_bounds")
            out_ref[k] = tb + pos

    @pl.when(pl.program_id(0) >= nsteps_tc)
    def _passthrough():
        out_ref[...] = sc_ref[...]


_sc_mesh = plsc.VectorSubcoreMesh(core_axis_name="c", subcore_axis_name="s")

_sc_cp = pltpu.CompilerParams()
if "needs_layout_passes" in pltpu.CompilerParams.__dataclass_fields__:
    _sc_cp = dataclasses.replace(_sc_cp, needs_layout_passes=False)


@functools.partial(
    pl.kernel,
    out_type=jax.ShapeDtypeStruct((_BSC, _N * _N), jnp.float32),
    mesh=_sc_mesh,
    compiler_params=_sc_cp,
    cost_estimate=pl.CostEstimate(
        flops=_BSC * _N * _N * 25,
        bytes_accessed=_BSC * _N * _N * 4,
        transcendentals=0,
    ),
    scratch_types=[
        pltpu.VMEM((_N,), jnp.int32),
        pltpu.VMEM((136,), jnp.float32),
        pltpu.VMEM((408,), jnp.float32),
        pltpu.VMEM((32,), jnp.int32),
        pltpu.VMEM((96,), jnp.int32),
        pltpu.VMEM((_N * _N,), jnp.float32),
        pltpu.VMEM((_N * _N,), jnp.float32),
        pltpu.SemaphoreType.DMA,
        pltpu.SemaphoreType.DMA,
    ],
)
def _sc_kernel(t_hbm, tsw_hbm, pos_hbm, base_hbm, th_hbm, out_hbm,
               t_v, tsw_v, pos_v, base_v, th_v, ov0, ov1, sem0, sem1):
    wid = lax.axis_index("s") * _SC_NC + lax.axis_index("c")
    bpw = _BSC // _SC_NW
    pltpu.sync_copy(tsw_hbm, tsw_v)
    pltpu.sync_copy(pos_hbm, pos_v)
    pltpu.sync_copy(base_hbm, base_v)
    pltpu.sync_copy(th_hbm, th_v)
    iota = lax.iota(jnp.int32, 16)
    btc_base = t_hbm.shape[0] - _BSC
    bufs = (ov0, ov1)
    sems = (sem0, sem1)
    handles = [None, None]
    b0 = wid * bpw
    j0s = list(range(0, 192, 16)) + [184]

    for k in range(bpw):
        ov = bufs[k % 2]
        if handles[k % 2] is not None:
            handles[k % 2].wait()
        b = b0 + k
        pltpu.sync_copy(t_hbm.at[btc_base + b], t_v)

        def row_fn(i, carry, ov=ov):
            si = jnp.minimum(i + 1, _N - 1)
            sv = plsc.load_gather(t_v, [jnp.full((16,), si, jnp.int32)])
            pbase = (_N - 1) - i
            obase = i * _N
            for j0 in j0s:
                tv = plsc.load_gather(t_v, [j0 + iota])
                d = sv - tv
                x = jnp.maximum(jnp.abs(d), 1)
                bits = plsc.bitcast(x.astype(jnp.float32), jnp.int32)
                eb = (bits >> 23) - 127
                mb = bits & 0x7FFFFF
                bidx = plsc.load_gather(base_v, [eb])
                c1 = jnp.where(mb >= plsc.load_gather(th_v, [eb]), 1, 0)
                c2 = jnp.where(mb >= plsc.load_gather(th_v, [eb + 32]), 1, 0)
                c3 = jnp.where(mb >= plsc.load_gather(th_v, [eb + 64]), 1, 0)
                val = plsc.load_gather(tsw_v, [bidx + c1 + c2 + c3])
                pj = plsc.load_gather(pos_v, [(pbase + j0) + iota])
                plsc.store_scatter(ov, [(obase + j0) + iota], val + pj)
            return carry

        lax.fori_loop(0, _N, row_fn, 0)
        handles[k % 2] = pltpu.async_copy(ov, out_hbm.at[b], sems[k % 2])

    for h in handles:
        if h is not None:
            h.wait()


@jax.jit
def kernel(inputs, ts_w, pos_biases):
    bsz, n = inputs.shape
    btc = bsz - _BSC

    tsw_pad = jnp.zeros((1, 128), jnp.float32).at[0, :128].set(ts_w[:128])
    pos_pad = jnp.zeros((1, 512), jnp.float32).at[0, :2 * n - 1].set(pos_biases)

    pos_mat = pl.pallas_call(
        _pos_body,
        grid=(1,),
        in_specs=[pl.BlockSpec((1, 512), lambda g: (0, 0))],
        out_specs=pl.BlockSpec((1, n, n), lambda g: (0, 0, 0)),
        out_shape=jax.ShapeDtypeStruct((1, n, n), jnp.float32),
    )(pos_pad)

    t3 = inputs.reshape(bsz, 1, n)
    s3 = jnp.concatenate([inputs[:, 1:], inputs[:, n - 1:n]], axis=1)
    s3 = s3.reshape(bsz, n, 1)

    tsw1d = jnp.zeros((136,), jnp.float32).at[:129].set(ts_w)
    pos1d = jnp.zeros((408,), jnp.float32).at[:2 * n - 1].set(pos_biases)
    out_sc_flat = _sc_kernel(
        inputs, tsw1d, pos1d,
        jnp.asarray(_BASE_TBL), jnp.asarray(_TH_TBL),
    )
    out_sc = out_sc_flat.reshape(_BSC, n, n)

    nsteps_tc = btc // _BB
    nsteps = bsz // _BB
    return pl.pallas_call(
        _main_body,
        grid=(nsteps,),
        in_specs=[
            pl.BlockSpec((_BB, 1, n), lambda b: (b, 0, 0)),
            pl.BlockSpec((_BB, n, 1), lambda b: (b, 0, 0)),
            pl.BlockSpec((1, 128), lambda b: (0, 0)),
            pl.BlockSpec((1, n, n), lambda b: (0, 0, 0)),
            pl.BlockSpec((_BB, n, n),
                         lambda b: (jnp.maximum(b - (1024 - _BSC) // _BB, 0), 0, 0)),
        ],
        out_specs=pl.BlockSpec((_BB, n, n), lambda b: (b, 0, 0)),
        out_shape=jax.ShapeDtypeStruct((bsz, n, n), jnp.float32),
        compiler_params=pltpu.CompilerParams(
            dimension_semantics=("arbitrary",),
        ),
    )(t3, s3, tsw_pad, pos_mat, out_sc)


# speedup vs baseline: 1.1216x; 1.1216x over previous
import dataclasses
import functools

import numpy as np
import jax
import jax.numpy as jnp
from jax import lax
from jax.experimental import pallas as pl
from jax.experimental.pallas import tpu as pltpu
from jax.experimental.pallas import tpu_sc as plsc

_BUCKET_SIZE = 0.301
_N = 200
_NP = 208
_BB = 16
_BSC = 32
_SC_NC = 2
_SC_NS = 16
_SC_NW = _SC_NC * _SC_NS


def _build_bucket_tables():
    bs = np.float32(_BUCKET_SIZE)

    def bucket_f32(x32):
        return np.trunc(np.log(np.asarray(x32, np.float32)) / bs).astype(np.int32)

    base = np.zeros(32, np.int32)
    th = np.full((3, 32), 0x7FFFFFFF, np.int32)
    for e in range(31):
        base[e] = bucket_f32(np.float32(2.0 ** e))
        for k in range(1, 4):
            target = int(base[e]) + k
            m = np.exp(0.301 * target) / 2.0 ** e
            if m >= 2.0:
                continue
            mb0 = int(np.floor((m - 1.0) * 2 ** 23))
            lo, hi = max(mb0 - 64, 0), min(mb0 + 64, 2 ** 23 - 1)
            mbs = np.arange(lo, hi + 1, dtype=np.int64)
            vals = (((e + 127) << 23) | mbs).astype(np.int32).view(np.float32)
            idx = np.nonzero(bucket_f32(vals) >= target)[0]
            if len(idx):
                th[k - 1, e] = int(mbs[idx[0]])
    return base, th.reshape(-1)


_BASE_TBL, _TH_TBL = _build_bucket_tables()


def _pos_body(pos_ref, out_ref):
    n = _N
    i = jax.lax.broadcasted_iota(jnp.int32, (n, n), 0)
    j = jax.lax.broadcasted_iota(jnp.int32, (n, n), 1)
    p = (n - 1) + j - i
    hi = p >> 7
    lo = p & 127
    posv = jnp.zeros((n, n), jnp.float32)
    for c in range(4):
        chunk = jnp.broadcast_to(pos_ref[0:1, c * 128:(c + 1) * 128], (n, 128))
        g = jnp.take_along_axis(chunk, lo, axis=1, mode="promise_in_bounds")
        posv = jnp.where(hi == c, g, posv)
    out_ref[0] = posv


def _main_body(t_ref, s_ref, tsw_ref, pos_ref, out_ref):
    n = _N
    table = jnp.broadcast_to(tsw_ref[0:1, :], (n, 128))
    pos = pos_ref[0]
    for k in range(_BB):
        t = t_ref[k]
        s = s_ref[k]
        diff = s - t
        x = jnp.maximum(jnp.abs(diff.astype(jnp.float32)), 1.0)
        b = (jnp.log(x) / _BUCKET_SIZE).astype(jnp.int32)
        tb = jnp.take_along_axis(table, b, axis=1, mode="promise_in_bounds")
        out_ref[k] = tb + pos


_sc_mesh = plsc.VectorSubcoreMesh(core_axis_name="c", subcore_axis_name="s")

_sc_cp = pltpu.CompilerParams()
if "needs_layout_passes" in pltpu.CompilerParams.__dataclass_fields__:
    _sc_cp = dataclasses.replace(_sc_cp, needs_layout_passes=False)


@functools.partial(
    pl.kernel,
    out_type=jax.ShapeDtypeStruct((_BSC, _N * _N), jnp.float32),
    mesh=_sc_mesh,
    compiler_params=_sc_cp,
    cost_estimate=pl.CostEstimate(
        flops=_BSC * _N * _N * 25,
        bytes_accessed=_BSC * _N * _N * 4,
        transcendentals=0,
    ),
    scratch_types=[
        pltpu.VMEM((_N,), jnp.int32),
        pltpu.VMEM((136,), jnp.float32),
        pltpu.VMEM((408,), jnp.float32),
        pltpu.VMEM((32,), jnp.int32),
        pltpu.VMEM((96,), jnp.int32),
        pltpu.VMEM((_N * _N,), jnp.float32),
        pltpu.VMEM((_N * _N,), jnp.float32),
        pltpu.SemaphoreType.DMA,
        pltpu.SemaphoreType.DMA,
    ],
)
def _sc_kernel(t_hbm, tsw_hbm, pos_hbm, base_hbm, th_hbm, out_hbm,
               t_v, tsw_v, pos_v, base_v, th_v, ov0, ov1, sem0, sem1):
    wid = lax.axis_index("s") * _SC_NC + lax.axis_index("c")
    bpw = _BSC // _SC_NW
    pltpu.sync_copy(tsw_hbm, tsw_v)
    pltpu.sync_copy(pos_hbm, pos_v)
    pltpu.sync_copy(base_hbm, base_v)
    pltpu.sync_copy(th_hbm, th_v)
    iota = lax.iota(jnp.int32, 16)
    btc_base = t_hbm.shape[0] - _BSC
    bufs = (ov0, ov1)
    sems = (sem0, sem1)
    handles = [None, None]
    b0 = wid * bpw
    j0s = list(range(0, 192, 16)) + [184]

    for k in range(bpw):
        ov = bufs[k % 2]
        if handles[k % 2] is not None:
            handles[k % 2].wait()
        b = b0 + k
        pltpu.sync_copy(t_hbm.at[btc_base + b], t_v)

        def row_fn(i, carry, ov=ov):
            si = jnp.minimum(i + 1, _N - 1)
            sv = plsc.load_gather(t_v, [jnp.full((16,), si, jnp.int32)])
            pbase = (_N - 1) - i
            obase = i * _N
            for j0 in j0s:
                tv = plsc.load_gather(t_v, [j0 + iota])
                d = sv - tv
                x = jnp.maximum(jnp.abs(d), 1)
                bits = plsc.bitcast(x.astype(jnp.float32), jnp.int32)
                eb = (bits >> 23) - 127
                mb = bits & 0x7FFFFF
                bidx = plsc.load_gather(base_v, [eb])
                c1 = jnp.where(mb >= plsc.load_gather(th_v, [eb]), 1, 0)
                c2 = jnp.where(mb >= plsc.load_gather(th_v, [eb + 32]), 1, 0)
                c3 = jnp.where(mb >= plsc.load_gather(th_v, [eb + 64]), 1, 0)
                val = plsc.load_gather(tsw_v, [bidx + c1 + c2 + c3])
                pj = plsc.load_gather(pos_v, [(pbase + j0) + iota])
                plsc.store_scatter(ov, [(obase + j0) + iota], val + pj)
            return carry

        lax.fori_loop(0, _N, row_fn, 0)
        handles[k % 2] = pltpu.async_copy(ov, out_hbm.at[b], sems[k % 2])

    for h in handles:
        if h is not None:
            h.wait()


@jax.jit
def kernel(inputs, ts_w, pos_biases):
    bsz, n = inputs.shape
    btc = bsz - _BSC

    tsw_pad = jnp.zeros((1, 128), jnp.float32).at[0, :128].set(ts_w[:128])
    pos_pad = jnp.zeros((1, 512), jnp.float32).at[0, :2 * n - 1].set(pos_biases)

    pos_mat = pl.pallas_call(
        _pos_body,
        grid=(1,),
        in_specs=[pl.BlockSpec((1, 512), lambda g: (0, 0))],
        out_specs=pl.BlockSpec((1, n, n), lambda g: (0, 0, 0)),
        out_shape=jax.ShapeDtypeStruct((1, n, n), jnp.float32),
    )(pos_pad)

    t3 = inputs.reshape(bsz, 1, n)
    s3 = jnp.concatenate([inputs[:, 1:], inputs[:, n - 1:n]], axis=1)
    s3 = s3.reshape(bsz, n, 1)

    tsw1d = jnp.zeros((136,), jnp.float32).at[:129].set(ts_w)
    pos1d = jnp.zeros((408,), jnp.float32).at[:2 * n - 1].set(pos_biases)
    out_sc_flat = _sc_kernel(
        inputs, tsw1d, pos1d,
        jnp.asarray(_BASE_TBL), jnp.asarray(_TH_TBL),
    )
    out_sc = out_sc_flat.reshape(_BSC, n, n)

    out_tc = pl.pallas_call(
        _main_body,
        grid=(btc // _BB,),
        in_specs=[
            pl.BlockSpec((_BB, 1, n), lambda b: (b, 0, 0)),
            pl.BlockSpec((_BB, n, 1), lambda b: (b, 0, 0)),
            pl.BlockSpec((1, 128), lambda b: (0, 0)),
            pl.BlockSpec((1, n, n), lambda b: (0, 0, 0)),
        ],
        out_specs=pl.BlockSpec((_BB, n, n), lambda b: (b, 0, 0)),
        out_shape=jax.ShapeDtypeStruct((bsz, n, n), jnp.float32),
        compiler_params=pltpu.CompilerParams(
            dimension_semantics=("arbitrary",),
        ),
    )(t3, s3, tsw_pad, pos_mat)

    return lax.dynamic_update_slice(out_tc, out_sc, (btc, 0, 0))


# speedup vs baseline: 1.1222x; 1.0005x over previous
import dataclasses
import functools

import numpy as np
import jax
import jax.numpy as jnp
from jax import lax
from jax.experimental import pallas as pl
from jax.experimental.pallas import tpu as pltpu
from jax.experimental.pallas import tpu_sc as plsc

_BUCKET_SIZE = 0.301
_N = 200
_BB = 16
_BSC = 32
_SC_NC = 2
_SC_NS = 16
_SC_NW = _SC_NC * _SC_NS


def _build_bucket_tables():
    bs = np.float32(_BUCKET_SIZE)

    def bucket_f32(x32):
        return np.trunc(np.log(np.asarray(x32, np.float32)) / bs).astype(np.int32)

    base = np.zeros(32, np.int32)
    th = np.full((3, 32), 0x7FFFFFFF, np.int32)
    for e in range(31):
        base[e] = bucket_f32(np.float32(2.0 ** e))
        for k in range(1, 4):
            target = int(base[e]) + k
            m = np.exp(0.301 * target) / 2.0 ** e
            if m >= 2.0:
                continue
            mb0 = int(np.floor((m - 1.0) * 2 ** 23))
            lo, hi = max(mb0 - 64, 0), min(mb0 + 64, 2 ** 23 - 1)
            mbs = np.arange(lo, hi + 1, dtype=np.int64)
            vals = (((e + 127) << 23) | mbs).astype(np.int32).view(np.float32)
            idx = np.nonzero(bucket_f32(vals) >= target)[0]
            if len(idx):
                th[k - 1, e] = int(mbs[idx[0]])
    return base, th.reshape(-1)


_BASE_TBL, _TH_TBL = _build_bucket_tables()


def _pos_body(pos_ref, out_ref):
    n = _N
    i = jax.lax.broadcasted_iota(jnp.int32, (n, n), 0)
    j = jax.lax.broadcasted_iota(jnp.int32, (n, n), 1)
    p = (n - 1) + j - i
    hi = p >> 7
    lo = p & 127
    posv = jnp.zeros((n, n), jnp.float32)
    for c in range(4):
        chunk = jnp.broadcast_to(pos_ref[0:1, c * 128:(c + 1) * 128], (n, 128))
        g = jnp.take_along_axis(chunk, lo, axis=1, mode="promise_in_bounds")
        posv = jnp.where(hi == c, g, posv)
    out_ref[0] = posv


def _main_body(t_ref, s_ref, tsw_ref, pos_ref, out_ref):
    n = _N
    table = jnp.broadcast_to(tsw_ref[0:1, :], (n, 128))
    pos = pos_ref[0]
    for k in range(_BB):
        t = t_ref[k]
        s = s_ref[k]
        diff = s - t
        x = jnp.maximum(jnp.abs(diff.astype(jnp.float32)), 1.0)
        b = (jnp.log(x) / _BUCKET_SIZE).astype(jnp.int32)
        tb = jnp.take_along_axis(table, b, axis=1, mode="promise_in_bounds")
        out_ref[k] = tb + pos


_sc_mesh = plsc.VectorSubcoreMesh(core_axis_name="c", subcore_axis_name="s")

_sc_cp = pltpu.CompilerParams()
if "needs_layout_passes" in pltpu.CompilerParams.__dataclass_fields__:
    _sc_cp = dataclasses.replace(_sc_cp, needs_layout_passes=False)


@functools.partial(
    pl.kernel,
    out_type=jax.ShapeDtypeStruct((_BSC, _N * _N), jnp.float32),
    mesh=_sc_mesh,
    compiler_params=_sc_cp,
    cost_estimate=pl.CostEstimate(
        flops=_BSC * _N * _N * 25,
        bytes_accessed=_BSC * _N * _N * 4,
        transcendentals=0,
    ),
    scratch_types=[
        pltpu.VMEM((_N,), jnp.int32),
        pltpu.VMEM((136,), jnp.float32),
        pltpu.VMEM((408,), jnp.float32),
        pltpu.VMEM((32,), jnp.int32),
        pltpu.VMEM((96,), jnp.int32),
        pltpu.VMEM((_N * _N,), jnp.float32),
        pltpu.VMEM((_N * _N,), jnp.float32),
        pltpu.SemaphoreType.DMA,
        pltpu.SemaphoreType.DMA,
    ],
)
def _sc_kernel(t_hbm, tsw_hbm, pos_hbm, base_hbm, th_hbm, out_hbm,
               t_v, tsw_v, pos_v, base_v, th_v, ov0, ov1, sem0, sem1):
    wid = lax.axis_index("s") * _SC_NC + lax.axis_index("c")
    bpw = _BSC // _SC_NW
    pltpu.sync_copy(tsw_hbm, tsw_v)
    pltpu.sync_copy(pos_hbm, pos_v)
    pltpu.sync_copy(base_hbm, base_v)
    pltpu.sync_copy(th_hbm, th_v)
    iota = lax.iota(jnp.int32, 16)
    btc_base = t_hbm.shape[0] - _BSC
    bufs = (ov0, ov1)
    sems = (sem0, sem1)
    handles = [None, None]
    b0 = wid * bpw
    j0s = list(range(0, 192, 16)) + [184]

    for k in range(bpw):
        ov = bufs[k % 2]
        if handles[k % 2] is not None:
            handles[k % 2].wait()
        b = b0 + k
        pltpu.sync_copy(t_hbm.at[btc_base + b], t_v)

        def row_fn(i, carry, ov=ov):
            si = jnp.minimum(i + 1, _N - 1)
            sv = plsc.load_gather(t_v, [jnp.full((16,), si, jnp.int32)])
            pbase = (_N - 1) - i
            obase = i * _N
            for j0 in j0s:
                tv = plsc.load_gather(t_v, [j0 + iota])
                d = sv - tv
                x = jnp.maximum(jnp.abs(d), 1)
                bits = plsc.bitcast(x.astype(jnp.float32), jnp.int32)
                eb = (bits >> 23) - 127
                mb = bits & 0x7FFFFF
                bidx = plsc.load_gather(base_v, [eb])
                c1 = jnp.where(mb >= plsc.load_gather(th_v, [eb]), 1, 0)
                c2 = jnp.where(mb >= plsc.load_gather(th_v, [eb + 32]), 1, 0)
                c3 = jnp.where(mb >= plsc.load_gather(th_v, [eb + 64]), 1, 0)
                val = plsc.load_gather(tsw_v, [bidx + c1 + c2 + c3])
                pj = plsc.load_gather(pos_v, [(pbase + j0) + iota])
                plsc.store_scatter(ov, [(obase + j0) + iota], val + pj)
            return carry

        lax.fori_loop(0, _N, row_fn, 0)
        handles[k % 2] = pltpu.async_copy(ov, out_hbm.at[b], sems[k % 2])

    for h in handles:
        if h is not None:
            h.wait()


@jax.jit
def kernel(inputs, ts_w, pos_biases):
    bsz, n = inputs.shape
    btc = bsz - _BSC

    tsw_pad = jnp.zeros((1, 128), jnp.float32).at[0, :128].set(ts_w[:128])
    pos_pad = jnp.zeros((1, 512), jnp.float32).at[0, :2 * n - 1].set(pos_biases)

    pos_mat = pl.pallas_call(
        _pos_body,
        grid=(1,),
        in_specs=[pl.BlockSpec((1, 512), lambda g: (0, 0))],
        out_specs=pl.BlockSpec((1, n, n), lambda g: (0, 0, 0)),
        out_shape=jax.ShapeDtypeStruct((1, n, n), jnp.float32),
    )(pos_pad)

    t3 = inputs.reshape(bsz, 1, n)
    s3 = jnp.concatenate([inputs[:, 1:], inputs[:, n - 1:n]], axis=1)
    s3 = s3.reshape(bsz, n, 1)

    tsw1d = jnp.zeros((136,), jnp.float32).at[:129].set(ts_w)
    pos1d = jnp.zeros((408,), jnp.float32).at[:2 * n - 1].set(pos_biases)
    out_sc_flat = _sc_kernel(
        inputs, tsw1d, pos1d,
        jnp.asarray(_BASE_TBL), jnp.asarray(_TH_TBL),
    )
    out_sc = out_sc_flat.reshape(_BSC, n, n)

    out_tc = pl.pallas_call(
        _main_body,
        grid=(btc // _BB,),
        in_specs=[
            pl.BlockSpec((_BB, 1, n), lambda b: (b, 0, 0)),
            pl.BlockSpec((_BB, n, 1), lambda b: (b, 0, 0)),
            pl.BlockSpec((1, 128), lambda b: (0, 0)),
            pl.BlockSpec((1, n, n), lambda b: (0, 0, 0)),
        ],
        out_specs=pl.BlockSpec((_BB, n, n), lambda b: (b, 0, 0)),
        out_shape=jax.ShapeDtypeStruct((bsz, n, n), jnp.float32),
        compiler_params=pltpu.CompilerParams(
            dimension_semantics=("parallel",),
        ),
    )(t3, s3, tsw_pad, pos_mat)

    return lax.dynamic_update_slice(out_tc, out_sc, (btc, 0, 0))
